# baseline (speedup 1.0000x reference)
import functools

import jax
import jax.numpy as jnp
from jax import lax
from jax.experimental import pallas as pl
from jax.experimental.pallas import tpu as pltpu
from jax.experimental.pallas import tpu_sc as plsc

N = 10000
E = 320000
F_IN = 128
DIM = 64
NUM_GRAPHS = 64

NPAD = 10240
NW = 32
CHUNK = 128
NB = 3
NCHUNK = 81
EPW = NCHUNK * CHUNK
RPT = NPAD // 16

_F32 = jnp.float32


def _mesh():
    return plsc.VectorSubcoreMesh(core_axis_name="c", subcore_axis_name="s")


@functools.partial(
    pl.kernel,
    mesh=_mesh(),
    compiler_params=pltpu.CompilerParams(needs_layout_passes=False, use_tc_tiling_on_sc=False),
    out_type=jax.ShapeDtypeStruct((2, NPAD), _F32),
    scratch_types=[
        pltpu.VMEM((NCHUNK, CHUNK), jnp.int32),
        pltpu.VMEM((NCHUNK, CHUNK), _F32),
        pltpu.VMEM((NPAD,), _F32),
        pltpu.VMEM_SHARED((16, NPAD), _F32),
        pltpu.VMEM((16, RPT), _F32),
        pltpu.VMEM((RPT,), _F32),
    ],
)
def _deg_kernel(dst_hbm, w_hbm, out_hbm, dstv, wv, priv, shared, buf, obuf):
    cid = lax.axis_index("c")
    sid = lax.axis_index("s")
    wid = cid * 16 + sid
    pltpu.sync_copy(dst_hbm.at[wid], dstv)
    pltpu.sync_copy(w_hbm.at[wid], wv)
    z16 = jnp.zeros((16,), _F32)

    def zero_body(i, carry):
        priv[pl.ds(i * 16, 16)] = z16
        return carry

    lax.fori_loop(0, NPAD // 16, zero_body, 0)

    def scat_body(r, carry):
        for c in range(CHUNK // 16):
            idx = dstv[r, pl.ds(c * 16, 16)]
            vals = wv[r, pl.ds(c * 16, 16)]
            plsc.addupdate_scatter(priv, [idx], vals)
        return carry

    lax.fori_loop(0, NCHUNK, scat_body, 0)
    pltpu.sync_copy(priv, shared.at[sid])
    plsc.subcore_barrier()
    pltpu.sync_copy(shared.at[:, pl.ds(sid * RPT, RPT)], buf)

    def red_body(k, carry):
        a = buf[0, pl.ds(k * 16, 16)]
        for t in range(1, 16):
            a = a + buf[t, pl.ds(k * 16, 16)]
        obuf[pl.ds(k * 16, 16)] = a
        return carry

    lax.fori_loop(0, RPT // 16, red_body, 0)
    pltpu.sync_copy(obuf, out_hbm.at[cid, pl.ds(sid * RPT, RPT)])


def _make_spmm(D):
    @functools.partial(
        pl.kernel,
        mesh=_mesh(),
        compiler_params=pltpu.CompilerParams(needs_layout_passes=False, use_tc_tiling_on_sc=False),
        out_type=jax.ShapeDtypeStruct((2, NPAD, D), _F32),
        scratch_types=[
            pltpu.VMEM((NCHUNK, CHUNK), jnp.int32),
            pltpu.VMEM((NCHUNK, CHUNK), jnp.int32),
            pltpu.VMEM((NCHUNK, CHUNK), _F32),
            [pltpu.VMEM((CHUNK, D), _F32)] * NB,
            [pltpu.VMEM((CHUNK, D), _F32)] * NB,
            pltpu.VMEM((CHUNK, D), _F32),
            pltpu.VMEM_SHARED((NPAD, D), _F32),
            [pltpu.SemaphoreType.DMA] * NB,
            [pltpu.SemaphoreType.DMA] * NB,
        ],
    )
    def spmm(src_hbm, dst_hbm, w_hbm, hp2_hbm, out_hbm,
             srcv, dstv, wv, gbuf, sbuf, zbuf, acc, semg, sems):
        cid = lax.axis_index("c")
        sid = lax.axis_index("s")
        wid = cid * 16 + sid
        hp_hbm = hp2_hbm.at[cid]
        z16 = jnp.zeros((16,), _F32)

        def zero_body(r, carry):
            for c in range(D // 16):
                zbuf[r, pl.ds(c * 16, 16)] = z16
            return carry

        lax.fori_loop(0, CHUNK, zero_body, 0)
        for q in range(RPT // CHUNK):
            pltpu.sync_copy(zbuf, acc.at[pl.ds(sid * RPT + q * CHUNK, CHUNK)])
        pltpu.sync_copy(src_hbm.at[wid], srcv)
        pltpu.sync_copy(dst_hbm.at[wid], dstv)
        pltpu.sync_copy(w_hbm.at[wid], wv)
        plsc.subcore_barrier()

        for b in range(NB):
            pltpu.async_copy(hp_hbm.at[srcv.at[b]], gbuf[b], semg[b])

        def step(t, carry):
            for b in range(NB):
                j = t * NB + b
                pltpu.make_async_copy(
                    hp_hbm.at[srcv.at[j]], gbuf[b], semg[b]).wait()

                @pl.when(t > 0)
                def _wait_prev_scatter():
                    pltpu.make_async_copy(
                        sbuf[b], acc.at[dstv.at[j]], sems[b]).wait()

                def scale_body(g, c2):
                    wvec = wv[j, pl.ds(g * 16, 16)]
                    for l in range(16):
                        e = g * 16 + l
                        wsc = wvec[l]
                        for c in range(D // 16):
                            sbuf[b][e, pl.ds(c * 16, 16)] = (
                                gbuf[b][e, pl.ds(c * 16, 16)] * wsc)
                    return c2

                lax.fori_loop(0, CHUNK // 16, scale_body, 0)

                @pl.when(t < NCHUNK // NB - 1)
                def _issue_next_gather():
                    pltpu.async_copy(
                        hp_hbm.at[srcv.at[j + NB]], gbuf[b], semg[b])

                pltpu.async_copy(sbuf[b], acc.at[dstv.at[j]], sems[b],
                                 add=True)
            return carry

        lax.fori_loop(0, NCHUNK // NB, step, 0)
        for b in range(NB):
            pltpu.make_async_copy(
                sbuf[b], acc.at[dstv.at[NCHUNK - NB + b]], sems[b]).wait()
        plsc.subcore_barrier()
        for q in range(RPT // CHUNK):
            pltpu.sync_copy(
                acc.at[pl.ds(sid * RPT + q * CHUNK, CHUNK)], zbuf)
            pltpu.sync_copy(
                zbuf, out_hbm.at[cid, pl.ds(sid * RPT + q * CHUNK, CHUNK)])

    return spmm


_spmm64 = _make_spmm(DIM)
_spmm16 = _make_spmm(16)


def _tc1_body(x_ref, w1_ref, dp0_ref, dp1_ref, hp_ref, dinv_ref):
    deg = dp0_ref[...] + dp1_ref[...] + 1.0
    dinv = jnp.where(deg > 0, lax.rsqrt(jnp.maximum(deg, 1e-12)), 0.0)
    h = jnp.dot(x_ref[...], w1_ref[...], preferred_element_type=_F32)
    hp_ref[...] = h * dinv
    dinv_ref[...] = dinv


def _tc_mid_body(a0_ref, a1_ref, hp_ref, dinv_ref, b_ref, g_ref, be_ref,
                 wn_ref, out_ref, bcast):
    dinv = dinv_ref[...]
    o = dinv * (a0_ref[...] + a1_ref[...] + hp_ref[...]) + b_ref[...]
    m = jnp.mean(o, axis=0, keepdims=True)
    v = jnp.mean((o - m) ** 2, axis=0, keepdims=True)
    on = g_ref[...] * (o - m) * lax.rsqrt(v + 1e-5) + be_ref[...]
    r = jnp.maximum(on, 0.0)
    h = jnp.dot(r, wn_ref[...], preferred_element_type=_F32) * dinv
    if bcast:
        out_ref[...] = jnp.broadcast_to(h, out_ref.shape)
    else:
        out_ref[...] = h


def _tc4_body(a0_ref, a1_ref, hp_ref, dinv_ref, b_ref, g_ref, be_ref,
              batch_ref, out_ref):
    o = dinv_ref[...] * (a0_ref[...] + a1_ref[...] + hp_ref[...]) + b_ref[...]
    m = jnp.mean(o)
    v = jnp.mean((o - m) ** 2)
    on = g_ref[...] * (o - m) * lax.rsqrt(v + 1e-5) + be_ref[...]
    logit = on / 5.0
    ids = lax.broadcasted_iota(jnp.int32, (N, NUM_GRAPHS), 1)
    oh = batch_ref[...] == ids
    mg = jnp.max(jnp.where(oh, logit, -1e30), axis=0, keepdims=True)
    mb = jnp.sum(jnp.where(oh, mg, 0.0), axis=1, keepdims=True)
    z = jnp.exp(logit - mb)
    sg = jnp.sum(jnp.where(oh, z, 0.0), axis=0, keepdims=True)
    sb = jnp.sum(jnp.where(oh, sg, 0.0), axis=1, keepdims=True)
    out_ref[...] = z / (sb + 1e-16)


def _sds(shape):
    return jax.ShapeDtypeStruct(shape, _F32)


def kernel(x, edge_index, edge_weight, batch,
           W1, b1, g1, be1, W2, b2, g2, be2, W3, b3, g3, be3):
    src = edge_index[0]
    dst = edge_index[1]
    pad = NW * EPW - E
    pad_dst = (jnp.arange(pad, dtype=jnp.int32) % (NPAD - N)) + N
    srcp = jnp.concatenate([src, jnp.zeros((pad,), jnp.int32)]).reshape(
        NW, NCHUNK, CHUNK)
    dstp = jnp.concatenate([dst, pad_dst]).reshape(
        NW, NCHUNK, CHUNK)
    wp = jnp.concatenate([edge_weight, jnp.zeros((pad,), _F32)]).reshape(
        NW, NCHUNK, CHUNK)

    degp = _deg_kernel(dstp, wp)
    dp0 = degp[0, :N].reshape(N, 1)
    dp1 = degp[1, :N].reshape(N, 1)

    hp1, dinv = pl.pallas_call(
        _tc1_body, out_shape=[_sds((N, DIM)), _sds((N, 1))],
    )(x, W1, dp0, dp1)

    hp1d = jnp.stack([hp1, hp1])
    acc1 = _spmm64(srcp, dstp, wp, hp1d)
    hp2 = pl.pallas_call(
        functools.partial(_tc_mid_body, bcast=False), out_shape=_sds((N, DIM)),
    )(acc1[0, :N], acc1[1, :N], hp1, dinv,
      b1.reshape(1, DIM), g1.reshape(1, DIM), be1.reshape(1, DIM), W2)

    acc2 = _spmm64(srcp, dstp, wp, jnp.stack([hp2, hp2]))
    hp3b = pl.pallas_call(
        functools.partial(_tc_mid_body, bcast=True), out_shape=_sds((N, 16)),
    )(acc2[0, :N], acc2[1, :N], hp2, dinv,
      b2.reshape(1, DIM), g2.reshape(1, DIM), be2.reshape(1, DIM), W3)

    acc3 = _spmm16(srcp, dstp, wp, jnp.stack([hp3b, hp3b]))
    out = pl.pallas_call(
        _tc4_body, out_shape=_sds((N, 1)),
    )(acc3[0, :N, :1], acc3[1, :N, :1], hp3b[:, :1], dinv,
      b3.reshape(1, 1), g3.reshape(1, 1), be3.reshape(1, 1),
      batch.reshape(N, 1))
    return out


# speedup vs baseline: 1.2390x; 1.2390x over previous
import functools

import jax
import jax.numpy as jnp
from jax import lax
from jax.experimental import pallas as pl
from jax.experimental.pallas import tpu as pltpu
from jax.experimental.pallas import tpu_sc as plsc

N = 10000
E = 320000
F_IN = 128
DIM = 64
NUM_GRAPHS = 64

NPAD = 10240
NW = 32
CHUNK = 128
NB = 2
TOTC = 2592
NCHUNK = TOTC // NW
C0 = 138
C1 = 24
C0TOT = 16 * C0
RPT = NPAD // 16

_F32 = jnp.float32


def _mesh():
    return plsc.VectorSubcoreMesh(core_axis_name="c", subcore_axis_name="s")


@functools.partial(
    pl.kernel,
    mesh=_mesh(),
    compiler_params=pltpu.CompilerParams(needs_layout_passes=False, use_tc_tiling_on_sc=False),
    out_type=jax.ShapeDtypeStruct((2, NPAD), _F32),
    scratch_types=[
        pltpu.VMEM((NCHUNK, CHUNK), jnp.int32),
        pltpu.VMEM((NCHUNK, CHUNK), _F32),
        pltpu.VMEM((NPAD,), _F32),
        pltpu.VMEM_SHARED((16, NPAD), _F32),
        pltpu.VMEM((16, RPT), _F32),
        pltpu.VMEM((RPT,), _F32),
    ],
)
def _deg_kernel(dst_hbm, w_hbm, out_hbm, dstv, wv, priv, shared, buf, obuf):
    cid = lax.axis_index("c")
    sid = lax.axis_index("s")
    wid = cid * 16 + sid
    pltpu.sync_copy(dst_hbm.at[pl.ds(wid * NCHUNK, NCHUNK)], dstv)
    pltpu.sync_copy(w_hbm.at[pl.ds(wid * NCHUNK, NCHUNK)], wv)
    z16 = jnp.zeros((16,), _F32)

    def zero_body(i, carry):
        priv[pl.ds(i * 16, 16)] = z16
        return carry

    lax.fori_loop(0, NPAD // 16, zero_body, 0)

    def scat_body(r, carry):
        for c in range(CHUNK // 16):
            idx = dstv[r, pl.ds(c * 16, 16)]
            vals = wv[r, pl.ds(c * 16, 16)]
            plsc.addupdate_scatter(priv, [idx], vals)
        return carry

    lax.fori_loop(0, NCHUNK, scat_body, 0)
    pltpu.sync_copy(priv, shared.at[sid])
    plsc.subcore_barrier()
    pltpu.sync_copy(shared.at[:, pl.ds(sid * RPT, RPT)], buf)

    def red_body(k, carry):
        a = buf[0, pl.ds(k * 16, 16)]
        for t in range(1, 16):
            a = a + buf[t, pl.ds(k * 16, 16)]
        obuf[pl.ds(k * 16, 16)] = a
        return carry

    lax.fori_loop(0, RPT // 16, red_body, 0)
    pltpu.sync_copy(obuf, out_hbm.at[cid, pl.ds(sid * RPT, RPT)])


def _make_spmm(D):
    @functools.partial(
        pl.kernel,
        mesh=_mesh(),
        compiler_params=pltpu.CompilerParams(needs_layout_passes=False, use_tc_tiling_on_sc=False),
        out_type=jax.ShapeDtypeStruct((2, NPAD, D), _F32),
        scratch_types=[
            pltpu.VMEM((C0, CHUNK), jnp.int32),
            pltpu.VMEM((C0, CHUNK), jnp.int32),
            pltpu.VMEM((C0, CHUNK), _F32),
            [pltpu.VMEM((CHUNK, D), _F32)] * NB,
            [pltpu.VMEM((CHUNK, D), _F32)] * NB,
            pltpu.VMEM_SHARED((NPAD, D), _F32),
            [pltpu.SemaphoreType.DMA] * NB,
            [pltpu.SemaphoreType.DMA] * NB,
        ],
    )
    def spmm(src_hbm, dst_hbm, w_hbm, hp_hbm, out_hbm,
             srcv, dstv, wv, gbuf, sbuf, acc, semg, sems):
        zbuf = sbuf[0]
        cid = lax.axis_index("c")
        sid = lax.axis_index("s")
        z16 = jnp.zeros((16,), _F32)

        def zero_body(r, carry):
            for c in range(D // 16):
                zbuf[r, pl.ds(c * 16, 16)] = z16
            return carry

        lax.fori_loop(0, CHUNK, zero_body, 0)
        for q in range(RPT // CHUNK):
            pltpu.sync_copy(zbuf, acc.at[pl.ds(sid * RPT + q * CHUNK, CHUNK)])
        plsc.subcore_barrier()

        def pipe(start, cnt):
            pltpu.sync_copy(src_hbm.at[pl.ds(start, cnt)],
                            srcv.at[pl.ds(0, cnt)])
            pltpu.sync_copy(dst_hbm.at[pl.ds(start, cnt)],
                            dstv.at[pl.ds(0, cnt)])
            pltpu.sync_copy(w_hbm.at[pl.ds(start, cnt)],
                            wv.at[pl.ds(0, cnt)])
            for b in range(NB):
                pltpu.async_copy(hp_hbm.at[srcv.at[b]], gbuf[b], semg[b])

            def step(t, carry):
                for b in range(NB):
                    j = t * NB + b
                    pltpu.make_async_copy(
                        hp_hbm.at[srcv.at[j]], gbuf[b], semg[b]).wait()

                    @pl.when(t > 0)
                    def _wait_prev_scatter():
                        pltpu.make_async_copy(
                            sbuf[b], acc.at[dstv.at[j]], sems[b]).wait()

                    def scale_body(g, c2):
                        wvec = wv[j, pl.ds(g * 16, 16)]
                        for l in range(16):
                            e = g * 16 + l
                            wsc = wvec[l]
                            for c in range(D // 16):
                                sbuf[b][e, pl.ds(c * 16, 16)] = (
                                    gbuf[b][e, pl.ds(c * 16, 16)] * wsc)
                        return c2

                    lax.fori_loop(0, CHUNK // 16, scale_body, 0)

                    @pl.when(t < cnt // NB - 1)
                    def _issue_next_gather():
                        pltpu.async_copy(
                            hp_hbm.at[srcv.at[j + NB]], gbuf[b], semg[b])

                    pltpu.async_copy(sbuf[b], acc.at[dstv.at[j]], sems[b],
                                     add=True)
                return carry

            lax.fori_loop(0, cnt // NB, step, 0)
            for b in range(NB):
                pltpu.make_async_copy(
                    sbuf[b], acc.at[dstv.at[cnt - NB + b]], sems[b]).wait()

        @pl.when(cid == 0)
        def _core0():
            pipe(sid * C0, C0)

        @pl.when(cid == 1)
        def _core1():
            pipe(C0TOT + sid * C1, C1)

        plsc.subcore_barrier()
        for q in range(RPT // CHUNK):
            pltpu.sync_copy(
                acc.at[pl.ds(sid * RPT + q * CHUNK, CHUNK)], zbuf)
            pltpu.sync_copy(
                zbuf, out_hbm.at[cid, pl.ds(sid * RPT + q * CHUNK, CHUNK)])

    return spmm


_spmm64 = _make_spmm(DIM)
_spmm16 = _make_spmm(16)


def _tc1_body(x_ref, w1_ref, dp0_ref, dp1_ref, hp_ref, dinv_ref):
    deg = dp0_ref[...] + dp1_ref[...] + 1.0
    dinv = jnp.where(deg > 0, lax.rsqrt(jnp.maximum(deg, 1e-12)), 0.0)
    h = jnp.dot(x_ref[...], w1_ref[...], preferred_element_type=_F32)
    hp_ref[...] = h * dinv
    dinv_ref[...] = dinv


def _tc_mid_body(a0_ref, a1_ref, hp_ref, dinv_ref, b_ref, g_ref, be_ref,
                 wn_ref, out_ref, bcast):
    dinv = dinv_ref[...]
    o = dinv * (a0_ref[...] + a1_ref[...] + hp_ref[...]) + b_ref[...]
    m = jnp.mean(o, axis=0, keepdims=True)
    v = jnp.mean((o - m) ** 2, axis=0, keepdims=True)
    on = g_ref[...] * (o - m) * lax.rsqrt(v + 1e-5) + be_ref[...]
    r = jnp.maximum(on, 0.0)
    h = jnp.dot(r, wn_ref[...], preferred_element_type=_F32) * dinv
    if bcast:
        out_ref[...] = jnp.broadcast_to(h, out_ref.shape)
    else:
        out_ref[...] = h


def _tc4_body(a0_ref, a1_ref, hp_ref, dinv_ref, b_ref, g_ref, be_ref,
              batch_ref, out_ref):
    o = dinv_ref[...] * (a0_ref[...] + a1_ref[...] + hp_ref[...]) + b_ref[...]
    m = jnp.mean(o)
    v = jnp.mean((o - m) ** 2)
    on = g_ref[...] * (o - m) * lax.rsqrt(v + 1e-5) + be_ref[...]
    logit = on / 5.0
    ids = lax.broadcasted_iota(jnp.int32, (N, NUM_GRAPHS), 1)
    oh = batch_ref[...] == ids
    mg = jnp.max(jnp.where(oh, logit, -1e30), axis=0, keepdims=True)
    mb = jnp.sum(jnp.where(oh, mg, 0.0), axis=1, keepdims=True)
    z = jnp.exp(logit - mb)
    sg = jnp.sum(jnp.where(oh, z, 0.0), axis=0, keepdims=True)
    sb = jnp.sum(jnp.where(oh, sg, 0.0), axis=1, keepdims=True)
    out_ref[...] = z / (sb + 1e-16)


def _sds(shape):
    return jax.ShapeDtypeStruct(shape, _F32)


def kernel(x, edge_index, edge_weight, batch,
           W1, b1, g1, be1, W2, b2, g2, be2, W3, b3, g3, be3):
    src = edge_index[0]
    dst = edge_index[1]
    pad = TOTC * CHUNK - E
    pad_dst = (jnp.arange(pad, dtype=jnp.int32) % (NPAD - N)) + N
    srcp = jnp.concatenate([src, jnp.zeros((pad,), jnp.int32)]).reshape(
        TOTC, CHUNK)
    dstp = jnp.concatenate([dst, pad_dst]).reshape(TOTC, CHUNK)
    wp = jnp.concatenate([edge_weight, jnp.zeros((pad,), _F32)]).reshape(
        TOTC, CHUNK)

    degp = _deg_kernel(dstp, wp)
    dp0 = degp[0, :N].reshape(N, 1)
    dp1 = degp[1, :N].reshape(N, 1)

    hp1, dinv = pl.pallas_call(
        _tc1_body, out_shape=[_sds((N, DIM)), _sds((N, 1))],
    )(x, W1, dp0, dp1)

    acc1 = _spmm64(srcp, dstp, wp, hp1)
    hp2 = pl.pallas_call(
        functools.partial(_tc_mid_body, bcast=False), out_shape=_sds((N, DIM)),
    )(acc1[0, :N], acc1[1, :N], hp1, dinv,
      b1.reshape(1, DIM), g1.reshape(1, DIM), be1.reshape(1, DIM), W2)

    acc2 = _spmm64(srcp, dstp, wp, hp2)
    hp3b = pl.pallas_call(
        functools.partial(_tc_mid_body, bcast=True), out_shape=_sds((N, 16)),
    )(acc2[0, :N], acc2[1, :N], hp2, dinv,
      b2.reshape(1, DIM), g2.reshape(1, DIM), be2.reshape(1, DIM), W3)

    acc3 = _spmm16(srcp, dstp, wp, hp3b)
    out = pl.pallas_call(
        _tc4_body, out_shape=_sds((N, 1)),
    )(acc3[0, :N, :1], acc3[1, :N, :1], hp3b[:, :1], dinv,
      b3.reshape(1, 1), g3.reshape(1, 1), be3.reshape(1, 1),
      batch.reshape(N, 1))
    return out


# speedup vs baseline: 1.5863x; 1.2803x over previous
import functools

import jax
import jax.numpy as jnp
from jax import lax
from jax.experimental import pallas as pl
from jax.experimental.pallas import tpu as pltpu
from jax.experimental.pallas import tpu_sc as plsc

N = 10000
E = 320000
F_IN = 128
DIM = 64
NUM_GRAPHS = 64

NPAD = 10240
NW = 32
CHUNK = 128
NB = 2
TOTC = 2624
NCHUNK = TOTC // NW
RPT = NPAD // 16
HPT = N // 16

_F32 = jnp.float32


def _mesh():
    return plsc.VectorSubcoreMesh(core_axis_name="c", subcore_axis_name="s")


@functools.partial(
    pl.kernel,
    mesh=_mesh(),
    compiler_params=pltpu.CompilerParams(needs_layout_passes=False, use_tc_tiling_on_sc=False),
    out_type=jax.ShapeDtypeStruct((2, NPAD), _F32),
    scratch_types=[
        pltpu.VMEM((NCHUNK, CHUNK), jnp.int32),
        pltpu.VMEM((NCHUNK, CHUNK), _F32),
        pltpu.VMEM((NPAD,), _F32),
        pltpu.VMEM_SHARED((16, NPAD), _F32),
        pltpu.VMEM((16, RPT), _F32),
        pltpu.VMEM((RPT,), _F32),
    ],
)
def _deg_kernel(dst_hbm, w_hbm, out_hbm, dstv, wv, priv, shared, buf, obuf):
    cid = lax.axis_index("c")
    sid = lax.axis_index("s")
    wid = cid * 16 + sid
    pltpu.sync_copy(dst_hbm.at[pl.ds(wid * NCHUNK, NCHUNK)], dstv)
    pltpu.sync_copy(w_hbm.at[pl.ds(wid * NCHUNK, NCHUNK)], wv)
    z16 = jnp.zeros((16,), _F32)

    def zero_body(i, carry):
        priv[pl.ds(i * 16, 16)] = z16
        return carry

    lax.fori_loop(0, NPAD // 16, zero_body, 0)

    def scat_body(r, carry):
        for c in range(CHUNK // 16):
            idx = dstv[r, pl.ds(c * 16, 16)]
            vals = wv[r, pl.ds(c * 16, 16)]
            plsc.addupdate_scatter(priv, [idx], vals)
        return carry

    lax.fori_loop(0, NCHUNK, scat_body, 0)
    pltpu.sync_copy(priv, shared.at[sid])
    plsc.subcore_barrier()
    pltpu.sync_copy(shared.at[:, pl.ds(sid * RPT, RPT)], buf)

    def red_body(k, carry):
        a = buf[0, pl.ds(k * 16, 16)]
        for t in range(1, 16):
            a = a + buf[t, pl.ds(k * 16, 16)]
        obuf[pl.ds(k * 16, 16)] = a
        return carry

    lax.fori_loop(0, RPT // 16, red_body, 0)
    pltpu.sync_copy(obuf, out_hbm.at[cid, pl.ds(sid * RPT, RPT)])


def _make_spmm(D):
    @functools.partial(
        pl.kernel,
        mesh=_mesh(),
        compiler_params=pltpu.CompilerParams(needs_layout_passes=False, use_tc_tiling_on_sc=False),
        out_type=jax.ShapeDtypeStruct((2, NPAD, D), _F32),
        scratch_types=[
            pltpu.VMEM((NCHUNK, CHUNK), jnp.int32),
            pltpu.VMEM((NCHUNK, CHUNK), jnp.int32),
            pltpu.VMEM((NCHUNK, CHUNK), _F32),
            [pltpu.VMEM((CHUNK, D), _F32)] * 2,
            pltpu.VMEM_SHARED((N, D), _F32),
            pltpu.VMEM_SHARED((NPAD, D), _F32),
            [pltpu.SemaphoreType.DMA] * 2,
            [pltpu.SemaphoreType.DMA] * 2,
            pltpu.SemaphoreType.DMA,
        ],
    )
    def spmm(src_hbm, dst_hbm, w_hbm, hp_hbm, out_hbm,
             srcv, dstv, wv, slot, hp_sp, acc, semg, sems, semz):
        cid = lax.axis_index("c")
        sid = lax.axis_index("s")
        wid = cid * 16 + sid
        start = wid * NCHUNK
        z16 = jnp.zeros((16,), _F32)

        def zero_body(r, carry):
            for c in range(D // 16):
                slot[0][r, pl.ds(c * 16, 16)] = z16
            return carry

        lax.fori_loop(0, CHUNK, zero_body, 0)
        for q in range(RPT // CHUNK):
            pltpu.sync_copy(slot[0],
                            acc.at[pl.ds(sid * RPT + q * CHUNK, CHUNK)])
        hbase = sid * HPT
        for q in range(4):
            pltpu.sync_copy(hp_hbm.at[pl.ds(hbase + q * CHUNK, CHUNK)],
                            slot[1])
            pltpu.sync_copy(slot[1], hp_sp.at[pl.ds(hbase + q * CHUNK, CHUNK)])
        tail = HPT - 4 * CHUNK
        pltpu.sync_copy(hp_hbm.at[pl.ds(hbase + 4 * CHUNK, tail)],
                        slot[1].at[pl.ds(0, tail)])
        pltpu.sync_copy(slot[1].at[pl.ds(0, tail)],
                        hp_sp.at[pl.ds(hbase + 4 * CHUNK, tail)])
        pltpu.sync_copy(src_hbm.at[pl.ds(start, NCHUNK)], srcv)
        pltpu.sync_copy(dst_hbm.at[pl.ds(start, NCHUNK)], dstv)
        pltpu.sync_copy(w_hbm.at[pl.ds(start, NCHUNK)], wv)
        plsc.subcore_barrier()

        pltpu.async_copy(hp_sp.at[srcv.at[0]], slot[0], semg[0])

        def scale_body_for(b, j):
            def scale_body(g, c2):
                wvec = wv[j, pl.ds(g * 16, 16)]
                for l in range(16):
                    e = g * 16 + l
                    wsc = wvec[l]
                    for c in range(D // 16):
                        slot[b][e, pl.ds(c * 16, 16)] = (
                            slot[b][e, pl.ds(c * 16, 16)] * wsc)
                return c2
            lax.fori_loop(0, CHUNK // 16, scale_body, 0)

        def step(t, carry):
            j0 = 2 * t
            j1 = j0 + 1
            pltpu.make_async_copy(hp_sp.at[srcv.at[j0]], slot[0],
                                  semg[0]).wait()

            @pl.when(t == 0)
            def _first_gather1():
                pltpu.async_copy(hp_sp.at[srcv.at[j1]], slot[1], semg[1])

            scale_body_for(0, j0)
            pltpu.async_copy(slot[0], acc.at[dstv.at[j0]], sems[0], add=True)
            pltpu.make_async_copy(hp_sp.at[srcv.at[j1]], slot[1],
                                  semg[1]).wait()
            scale_body_for(1, j1)
            pltpu.async_copy(slot[1], acc.at[dstv.at[j1]], sems[1], add=True)
            @pl.when(t < NCHUNK // 2 - 1)
            def _refill():
                pltpu.make_async_copy(slot[0], acc.at[dstv.at[j0]],
                                      sems[0]).wait()
                pltpu.async_copy(hp_sp.at[srcv.at[j0 + 2]], slot[0], semg[0])
                pltpu.make_async_copy(slot[1], acc.at[dstv.at[j1]],
                                      sems[1]).wait()
                pltpu.async_copy(hp_sp.at[srcv.at[j1 + 2]], slot[1], semg[1])
            return carry

        lax.fori_loop(0, NCHUNK // 2, step, 0)
        for b in range(2):
            pltpu.make_async_copy(slot[b], acc.at[dstv.at[NCHUNK - 2 + b]],
                                  sems[b]).wait()
        plsc.subcore_barrier()
        for q in range(RPT // CHUNK):
            pltpu.sync_copy(
                acc.at[pl.ds(sid * RPT + q * CHUNK, CHUNK)], slot[0])
            pltpu.sync_copy(
                slot[0], out_hbm.at[cid, pl.ds(sid * RPT + q * CHUNK, CHUNK)])

    return spmm


_spmm64 = _make_spmm(DIM)
_spmm16 = _make_spmm(16)


def _tc1_body(x_ref, w1_ref, dp0_ref, dp1_ref, hp_ref, dinv_ref):
    deg = dp0_ref[...] + dp1_ref[...] + 1.0
    dinv = jnp.where(deg > 0, lax.rsqrt(jnp.maximum(deg, 1e-12)), 0.0)
    h = jnp.dot(x_ref[...], w1_ref[...], preferred_element_type=_F32)
    hp_ref[...] = h * dinv
    dinv_ref[...] = dinv


def _tc_mid_body(a0_ref, a1_ref, hp_ref, dinv_ref, b_ref, g_ref, be_ref,
                 wn_ref, out_ref, bcast):
    dinv = dinv_ref[...]
    o = dinv * (a0_ref[...] + a1_ref[...] + hp_ref[...]) + b_ref[...]
    m = jnp.mean(o, axis=0, keepdims=True)
    v = jnp.mean((o - m) ** 2, axis=0, keepdims=True)
    on = g_ref[...] * (o - m) * lax.rsqrt(v + 1e-5) + be_ref[...]
    r = jnp.maximum(on, 0.0)
    h = jnp.dot(r, wn_ref[...], preferred_element_type=_F32) * dinv
    if bcast:
        out_ref[...] = jnp.broadcast_to(h, out_ref.shape)
    else:
        out_ref[...] = h


def _tc4_body(a0_ref, a1_ref, hp_ref, dinv_ref, b_ref, g_ref, be_ref,
              batch_ref, out_ref):
    o = dinv_ref[...] * (a0_ref[...] + a1_ref[...] + hp_ref[...]) + b_ref[...]
    m = jnp.mean(o)
    v = jnp.mean((o - m) ** 2)
    on = g_ref[...] * (o - m) * lax.rsqrt(v + 1e-5) + be_ref[...]
    logit = on / 5.0
    ids = lax.broadcasted_iota(jnp.int32, (N, NUM_GRAPHS), 1)
    oh = batch_ref[...] == ids
    mg = jnp.max(jnp.where(oh, logit, -1e30), axis=0, keepdims=True)
    mb = jnp.sum(jnp.where(oh, mg, 0.0), axis=1, keepdims=True)
    z = jnp.exp(logit - mb)
    sg = jnp.sum(jnp.where(oh, z, 0.0), axis=0, keepdims=True)
    sb = jnp.sum(jnp.where(oh, sg, 0.0), axis=1, keepdims=True)
    out_ref[...] = z / (sb + 1e-16)


def _sds(shape):
    return jax.ShapeDtypeStruct(shape, _F32)


def kernel(x, edge_index, edge_weight, batch,
           W1, b1, g1, be1, W2, b2, g2, be2, W3, b3, g3, be3):
    src = edge_index[0]
    dst = edge_index[1]
    pad = TOTC * CHUNK - E
    pad_dst = (jnp.arange(pad, dtype=jnp.int32) % (NPAD - N)) + N
    srcp = jnp.concatenate([src, jnp.zeros((pad,), jnp.int32)]).reshape(
        TOTC, CHUNK)
    dstp = jnp.concatenate([dst, pad_dst]).reshape(TOTC, CHUNK)
    wp = jnp.concatenate([edge_weight, jnp.zeros((pad,), _F32)]).reshape(
        TOTC, CHUNK)

    degp = _deg_kernel(dstp, wp)
    dp0 = degp[0, :N].reshape(N, 1)
    dp1 = degp[1, :N].reshape(N, 1)

    hp1, dinv = pl.pallas_call(
        _tc1_body, out_shape=[_sds((N, DIM)), _sds((N, 1))],
    )(x, W1, dp0, dp1)

    acc1 = _spmm64(srcp, dstp, wp, hp1)
    hp2 = pl.pallas_call(
        functools.partial(_tc_mid_body, bcast=False), out_shape=_sds((N, DIM)),
    )(acc1[0, :N], acc1[1, :N], hp1, dinv,
      b1.reshape(1, DIM), g1.reshape(1, DIM), be1.reshape(1, DIM), W2)

    acc2 = _spmm64(srcp, dstp, wp, hp2)
    hp3b = pl.pallas_call(
        functools.partial(_tc_mid_body, bcast=True), out_shape=_sds((N, 16)),
    )(acc2[0, :N], acc2[1, :N], hp2, dinv,
      b2.reshape(1, DIM), g2.reshape(1, DIM), be2.reshape(1, DIM), W3)

    acc3 = _spmm16(srcp, dstp, wp, hp3b)
    out = pl.pallas_call(
        _tc4_body, out_shape=_sds((N, 1)),
    )(acc3[0, :N, :1], acc3[1, :N, :1], hp3b[:, :1], dinv,
      b3.reshape(1, 1), g3.reshape(1, 1), be3.reshape(1, 1),
      batch.reshape(N, 1))
    return out


# speedup vs baseline: 1.6850x; 1.0623x over previous
import functools

import jax
import jax.numpy as jnp
from jax import lax
from jax.experimental import pallas as pl
from jax.experimental.pallas import tpu as pltpu
from jax.experimental.pallas import tpu_sc as plsc

N = 10000
E = 320000
F_IN = 128
DIM = 64
NUM_GRAPHS = 64

NPAD = 10240
NW = 32
CHUNK = 80
TOTC = 4224
NCHUNK = TOTC // NW
RPT = NPAD // 16
HPT = N // 16

_F32 = jnp.float32


def _mesh():
    return plsc.VectorSubcoreMesh(core_axis_name="c", subcore_axis_name="s")


@functools.partial(
    pl.kernel,
    mesh=_mesh(),
    compiler_params=pltpu.CompilerParams(needs_layout_passes=False, use_tc_tiling_on_sc=False),
    out_type=jax.ShapeDtypeStruct((2, NPAD), _F32),
    scratch_types=[
        pltpu.VMEM((NCHUNK, CHUNK), jnp.int32),
        pltpu.VMEM((NCHUNK, CHUNK), _F32),
        pltpu.VMEM((NPAD,), _F32),
        pltpu.VMEM_SHARED((16, NPAD), _F32),
        pltpu.VMEM((16, RPT), _F32),
        pltpu.VMEM((RPT,), _F32),
    ],
)
def _deg_kernel(dst_hbm, w_hbm, out_hbm, dstv, wv, priv, shared, buf, obuf):
    cid = lax.axis_index("c")
    sid = lax.axis_index("s")
    wid = cid * 16 + sid
    pltpu.sync_copy(dst_hbm.at[pl.ds(wid * NCHUNK, NCHUNK)], dstv)
    pltpu.sync_copy(w_hbm.at[pl.ds(wid * NCHUNK, NCHUNK)], wv)
    z16 = jnp.zeros((16,), _F32)

    def zero_body(i, carry):
        priv[pl.ds(i * 16, 16)] = z16
        return carry

    lax.fori_loop(0, NPAD // 16, zero_body, 0)

    def scat_body(r, carry):
        for c in range(CHUNK // 16):
            idx = dstv[r, pl.ds(c * 16, 16)]
            vals = wv[r, pl.ds(c * 16, 16)]
            plsc.addupdate_scatter(priv, [idx], vals)
        return carry

    lax.fori_loop(0, NCHUNK, scat_body, 0)
    pltpu.sync_copy(priv, shared.at[sid])
    plsc.subcore_barrier()
    pltpu.sync_copy(shared.at[:, pl.ds(sid * RPT, RPT)], buf)

    def red_body(k, carry):
        a = buf[0, pl.ds(k * 16, 16)]
        for t in range(1, 16):
            a = a + buf[t, pl.ds(k * 16, 16)]
        obuf[pl.ds(k * 16, 16)] = a
        return carry

    lax.fori_loop(0, RPT // 16, red_body, 0)
    pltpu.sync_copy(obuf, out_hbm.at[cid, pl.ds(sid * RPT, RPT)])


def _make_spmm(D):
    @functools.partial(
        pl.kernel,
        mesh=_mesh(),
        compiler_params=pltpu.CompilerParams(needs_layout_passes=False, use_tc_tiling_on_sc=False),
        out_type=jax.ShapeDtypeStruct((2, NPAD, D), _F32),
        scratch_types=[
            pltpu.VMEM((NCHUNK, CHUNK), jnp.int32),
            pltpu.VMEM((NCHUNK, CHUNK), jnp.int32),
            pltpu.VMEM((NCHUNK, CHUNK), _F32),
            [pltpu.VMEM((CHUNK, D), _F32)] * 3,
            pltpu.VMEM_SHARED((N, D), _F32),
            pltpu.VMEM_SHARED((NPAD, D), _F32),
            [pltpu.SemaphoreType.DMA] * 3,
            [pltpu.SemaphoreType.DMA] * 3,
        ],
    )
    def spmm(src_hbm, dst_hbm, w_hbm, hp_hbm, out_hbm,
             srcv, dstv, wv, slot, hp_sp, acc, semg, sems):
        cid = lax.axis_index("c")
        sid = lax.axis_index("s")
        wid = cid * 16 + sid
        start = wid * NCHUNK
        z16 = jnp.zeros((16,), _F32)

        def zero_body(r, carry):
            for c in range(D // 16):
                slot[0][r, pl.ds(c * 16, 16)] = z16
            return carry

        lax.fori_loop(0, CHUNK, zero_body, 0)
        for q in range(RPT // CHUNK):
            pltpu.sync_copy(slot[0],
                            acc.at[pl.ds(sid * RPT + q * CHUNK, CHUNK)])
        hbase = sid * HPT
        nh = HPT // CHUNK
        for q in range(nh):
            pltpu.sync_copy(hp_hbm.at[pl.ds(hbase + q * CHUNK, CHUNK)],
                            slot[1])
            pltpu.sync_copy(slot[1], hp_sp.at[pl.ds(hbase + q * CHUNK, CHUNK)])
        tail = HPT - nh * CHUNK
        pltpu.sync_copy(hp_hbm.at[pl.ds(hbase + nh * CHUNK, tail)],
                        slot[1].at[pl.ds(0, tail)])
        pltpu.sync_copy(slot[1].at[pl.ds(0, tail)],
                        hp_sp.at[pl.ds(hbase + nh * CHUNK, tail)])
        pltpu.sync_copy(src_hbm.at[pl.ds(start, NCHUNK)], srcv)
        pltpu.sync_copy(dst_hbm.at[pl.ds(start, NCHUNK)], dstv)
        pltpu.sync_copy(w_hbm.at[pl.ds(start, NCHUNK)], wv)
        plsc.subcore_barrier()

        def scale_body_for(b, j):
            def scale_body(g, c2):
                wvec = wv[j, pl.ds(g * 16, 16)]
                for l in range(16):
                    e = g * 16 + l
                    wsc = wvec[l]
                    for c in range(D // 16):
                        slot[b][e, pl.ds(c * 16, 16)] = (
                            slot[b][e, pl.ds(c * 16, 16)] * wsc)
                return c2
            lax.fori_loop(0, CHUNK // 16, scale_body, 0)

        pltpu.async_copy(hp_sp.at[srcv.at[0]], slot[0], semg[0])
        pltpu.async_copy(hp_sp.at[srcv.at[1]], slot[1], semg[1])

        def step(t, carry):
            for p in range(3):
                k = 3 * t + p
                sl = p
                nsl = (p + 2) % 3
                pltpu.make_async_copy(hp_sp.at[srcv.at[k]], slot[sl],
                                      semg[sl]).wait()
                scale_body_for(sl, k)
                pltpu.async_copy(slot[sl], acc.at[dstv.at[k]], sems[sl],
                                 add=True)
                if p == 0:
                    @pl.when(t > 0)
                    def _drain0():
                        pltpu.make_async_copy(
                            slot[nsl], acc.at[dstv.at[k - 1]],
                            sems[nsl]).wait()

                    pltpu.async_copy(hp_sp.at[srcv.at[k + 2]], slot[nsl],
                                     semg[nsl])
                else:
                    pltpu.make_async_copy(
                        slot[nsl], acc.at[dstv.at[k - 1]], sems[nsl]).wait()

                    @pl.when(t < NCHUNK // 3 - 1)
                    def _refill():
                        pltpu.async_copy(hp_sp.at[srcv.at[k + 2]], slot[nsl],
                                         semg[nsl])
            return carry

        lax.fori_loop(0, NCHUNK // 3, step, 0)
        pltpu.make_async_copy(slot[2], acc.at[dstv.at[NCHUNK - 1]],
                              sems[2]).wait()
        plsc.subcore_barrier()
        for q in range(RPT // CHUNK):
            pltpu.sync_copy(
                acc.at[pl.ds(sid * RPT + q * CHUNK, CHUNK)], slot[0])
            pltpu.sync_copy(
                slot[0], out_hbm.at[cid, pl.ds(sid * RPT + q * CHUNK, CHUNK)])

    return spmm


_spmm64 = _make_spmm(DIM)
_spmm16 = _make_spmm(16)


def _tc1_body(x_ref, w1_ref, dp0_ref, dp1_ref, hp_ref, dinv_ref):
    deg = dp0_ref[...] + dp1_ref[...] + 1.0
    dinv = jnp.where(deg > 0, lax.rsqrt(jnp.maximum(deg, 1e-12)), 0.0)
    h = jnp.dot(x_ref[...], w1_ref[...], preferred_element_type=_F32)
    hp_ref[...] = h * dinv
    dinv_ref[...] = dinv


def _tc_mid_body(a0_ref, a1_ref, hp_ref, dinv_ref, b_ref, g_ref, be_ref,
                 wn_ref, out_ref, bcast):
    dinv = dinv_ref[...]
    o = dinv * (a0_ref[...] + a1_ref[...] + hp_ref[...]) + b_ref[...]
    m = jnp.mean(o, axis=0, keepdims=True)
    v = jnp.mean((o - m) ** 2, axis=0, keepdims=True)
    on = g_ref[...] * (o - m) * lax.rsqrt(v + 1e-5) + be_ref[...]
    r = jnp.maximum(on, 0.0)
    h = jnp.dot(r, wn_ref[...], preferred_element_type=_F32) * dinv
    if bcast:
        out_ref[...] = jnp.broadcast_to(h, out_ref.shape)
    else:
        out_ref[...] = h


def _tc4_body(a0_ref, a1_ref, hp_ref, dinv_ref, b_ref, g_ref, be_ref,
              batch_ref, out_ref):
    o = dinv_ref[...] * (a0_ref[...] + a1_ref[...] + hp_ref[...]) + b_ref[...]
    m = jnp.mean(o)
    v = jnp.mean((o - m) ** 2)
    on = g_ref[...] * (o - m) * lax.rsqrt(v + 1e-5) + be_ref[...]
    logit = on / 5.0
    ids = lax.broadcasted_iota(jnp.int32, (N, NUM_GRAPHS), 1)
    oh = batch_ref[...] == ids
    mg = jnp.max(jnp.where(oh, logit, -1e30), axis=0, keepdims=True)
    mb = jnp.sum(jnp.where(oh, mg, 0.0), axis=1, keepdims=True)
    z = jnp.exp(logit - mb)
    sg = jnp.sum(jnp.where(oh, z, 0.0), axis=0, keepdims=True)
    sb = jnp.sum(jnp.where(oh, sg, 0.0), axis=1, keepdims=True)
    out_ref[...] = z / (sb + 1e-16)


def _sds(shape):
    return jax.ShapeDtypeStruct(shape, _F32)


def kernel(x, edge_index, edge_weight, batch,
           W1, b1, g1, be1, W2, b2, g2, be2, W3, b3, g3, be3):
    src = edge_index[0]
    dst = edge_index[1]
    pad = TOTC * CHUNK - E
    pad_dst = (jnp.arange(pad, dtype=jnp.int32) % (NPAD - N)) + N
    srcp = jnp.concatenate([src, jnp.zeros((pad,), jnp.int32)]).reshape(
        TOTC, CHUNK)
    dstp = jnp.concatenate([dst, pad_dst]).reshape(TOTC, CHUNK)
    wp = jnp.concatenate([edge_weight, jnp.zeros((pad,), _F32)]).reshape(
        TOTC, CHUNK)

    degp = _deg_kernel(dstp, wp)
    dp0 = degp[0, :N].reshape(N, 1)
    dp1 = degp[1, :N].reshape(N, 1)

    hp1, dinv = pl.pallas_call(
        _tc1_body, out_shape=[_sds((N, DIM)), _sds((N, 1))],
    )(x, W1, dp0, dp1)

    acc1 = _spmm64(srcp, dstp, wp, hp1)
    hp2 = pl.pallas_call(
        functools.partial(_tc_mid_body, bcast=False), out_shape=_sds((N, DIM)),
    )(acc1[0, :N], acc1[1, :N], hp1, dinv,
      b1.reshape(1, DIM), g1.reshape(1, DIM), be1.reshape(1, DIM), W2)

    acc2 = _spmm64(srcp, dstp, wp, hp2)
    hp3b = pl.pallas_call(
        functools.partial(_tc_mid_body, bcast=True), out_shape=_sds((N, 16)),
    )(acc2[0, :N], acc2[1, :N], hp2, dinv,
      b2.reshape(1, DIM), g2.reshape(1, DIM), be2.reshape(1, DIM), W3)

    acc3 = _spmm16(srcp, dstp, wp, hp3b)
    out = pl.pallas_call(
        _tc4_body, out_shape=_sds((N, 1)),
    )(acc3[0, :N, :1], acc3[1, :N, :1], hp3b[:, :1], dinv,
      b3.reshape(1, 1), g3.reshape(1, 1), be3.reshape(1, 1),
      batch.reshape(N, 1))
    return out
